# baseline (speedup 1.0000x reference)
import functools
import jax
import jax.numpy as jnp
from jax import lax
from jax.experimental import pallas as pl
from jax.experimental.pallas import tpu as pltpu
from jax.experimental.pallas import tpu_sc as plsc

SEQ = 4096
DM = 1024
BATCH = 4
NW = 32
ROWS_PER_W = SEQ // NW
CH = 64


def _sc_body(table_hbm, out_hbm, buf, sem):
    w = lax.axis_index("s") * 2 + lax.axis_index("c")
    base = w * ROWS_PER_W
    for ci in range(ROWS_PER_W // CH):
        r = base + ci * CH
        pltpu.sync_copy(table_hbm.at[pl.ds(r, CH), :], buf)
        cps = [pltpu.make_async_copy(buf, out_hbm.at[b, pl.ds(r, CH), :], sem)
               for b in range(BATCH)]
        for c in cps:
            c.start()
        for c in cps:
            c.wait()


def kernel(input_ids, pos_table):
    mesh = plsc.VectorSubcoreMesh(core_axis_name="c", subcore_axis_name="s")
    k = functools.partial(
        pl.kernel,
        mesh=mesh,
        out_type=jax.ShapeDtypeStruct((BATCH, SEQ, DM), pos_table.dtype),
        scratch_types=[
            pltpu.VMEM((CH, DM), pos_table.dtype),
            pltpu.SemaphoreType.DMA,
        ],
    )(_sc_body)
    return k(pos_table)
